# baseline (speedup 1.0000x reference)
import functools

import jax
import jax.numpy as jnp
from jax.experimental import pallas as pl
from jax.experimental.pallas import tpu as pltpu


def _adapter_body(idx_ref, x_ref, dw_ref, db_ref, uw_ref, o_ref):
    x = x_ref[0]
    dw = dw_ref[0, 0]
    db = db_ref[0, 0, 0]
    uw = uw_ref[0, 0]
    z = jnp.dot(x, dw, preferred_element_type=jnp.float32) + db[None, :]
    z = z * jax.nn.sigmoid(z)
    o_ref[0, 0] = jnp.dot(z, uw, preferred_element_type=jnp.float32)


@jax.jit
def kernel(x, expert_index, down_w, down_b, up_w):
    B, S, C = x.shape
    M, N, _, D = down_w.shape
    TS = 256
    s_blocks = S // TS

    idx = expert_index.astype(jnp.int32).reshape(M * B)
    down_b4 = down_b.reshape(M, N, 1, D)

    grid = (M, B, s_blocks)
    grid_spec = pltpu.PrefetchScalarGridSpec(
        num_scalar_prefetch=1,
        grid=grid,
        in_specs=[
            pl.BlockSpec((1, TS, C), lambda m, b, s, idx_ref: (b, s, 0)),
            pl.BlockSpec(
                (1, 1, C, D),
                lambda m, b, s, idx_ref: (m, idx_ref[m * B + b], 0, 0),
            ),
            pl.BlockSpec(
                (1, 1, 1, D),
                lambda m, b, s, idx_ref: (m, idx_ref[m * B + b], 0, 0),
            ),
            pl.BlockSpec(
                (1, 1, D, C),
                lambda m, b, s, idx_ref: (m, idx_ref[m * B + b], 0, 0),
            ),
        ],
        out_specs=pl.BlockSpec(
            (1, 1, TS, C), lambda m, b, s, idx_ref: (m, b, s, 0)
        ),
    )

    out = pl.pallas_call(
        functools.partial(_adapter_body),
        grid_spec=grid_spec,
        out_shape=jax.ShapeDtypeStruct((M, B, S, C), jnp.float32),
        compiler_params=pltpu.CompilerParams(
            dimension_semantics=("parallel", "parallel", "parallel"),
        ),
    )(idx, x, down_w, down_b4, up_w)
    return out


# speedup vs baseline: 1.1343x; 1.1343x over previous
import functools

import jax
import jax.numpy as jnp
from jax.experimental import pallas as pl
from jax.experimental.pallas import tpu as pltpu


def _adapter_body(idx_ref, x_ref, dw_ref, db_ref, uw_ref, o_ref):
    x = x_ref[0]
    dw = dw_ref[0, 0]
    db = db_ref[0, 0, 0]
    uw = uw_ref[0, 0]
    z = jnp.dot(x, dw, preferred_element_type=jnp.float32) + db[None, :]
    z = z * jax.nn.sigmoid(z)
    o_ref[0, 0] = jnp.dot(z, uw, preferred_element_type=jnp.float32)


@jax.jit
def kernel(x, expert_index, down_w, down_b, up_w):
    B, S, C = x.shape
    M, N, _, D = down_w.shape
    TS = 1024
    s_blocks = S // TS

    idx = expert_index.astype(jnp.int32).reshape(M * B)
    down_b4 = down_b.reshape(M, N, 1, D)

    grid = (M, B, s_blocks)
    grid_spec = pltpu.PrefetchScalarGridSpec(
        num_scalar_prefetch=1,
        grid=grid,
        in_specs=[
            pl.BlockSpec((1, TS, C), lambda m, b, s, idx_ref: (b, s, 0)),
            pl.BlockSpec(
                (1, 1, C, D),
                lambda m, b, s, idx_ref: (m, idx_ref[m * B + b], 0, 0),
            ),
            pl.BlockSpec(
                (1, 1, 1, D),
                lambda m, b, s, idx_ref: (m, idx_ref[m * B + b], 0, 0),
            ),
            pl.BlockSpec(
                (1, 1, D, C),
                lambda m, b, s, idx_ref: (m, idx_ref[m * B + b], 0, 0),
            ),
        ],
        out_specs=pl.BlockSpec(
            (1, 1, TS, C), lambda m, b, s, idx_ref: (m, b, s, 0)
        ),
    )

    out = pl.pallas_call(
        functools.partial(_adapter_body),
        grid_spec=grid_spec,
        out_shape=jax.ShapeDtypeStruct((M, B, S, C), jnp.float32),
        compiler_params=pltpu.CompilerParams(
            dimension_semantics=("parallel", "parallel", "parallel"),
        ),
    )(idx, x, down_w, down_b4, up_w)
    return out


# speedup vs baseline: 1.5041x; 1.3260x over previous
import functools

import jax
import jax.numpy as jnp
from jax.experimental import pallas as pl
from jax.experimental.pallas import tpu as pltpu


def _adapter_body(x_ref, dw_ref, db_ref, uw_ref, o_ref):
    x = x_ref[0]
    dw = dw_ref[0, 0]
    db = db_ref[0, 0, 0]
    uw = uw_ref[0, 0]
    z = jnp.dot(x, dw, preferred_element_type=jnp.float32) + db[None, :]
    z = z * jax.nn.sigmoid(z)
    o_ref[0, 0] = jnp.dot(z, uw, preferred_element_type=jnp.float32)


@jax.jit
def kernel(x, expert_index, down_w, down_b, up_w):
    B, S, C = x.shape
    M, N, _, D = down_w.shape
    TS = 512
    s_blocks = S // TS

    idx = expert_index.astype(jnp.int32)
    m = jnp.arange(M)[:, None]
    bdw = down_w[m, idx]
    bdb = down_b[m, idx].reshape(M, B, 1, D)
    buw = up_w[m, idx]

    grid = (M, B, s_blocks)

    out = pl.pallas_call(
        _adapter_body,
        grid=grid,
        in_specs=[
            pl.BlockSpec((1, TS, C), lambda mm, b, s: (b, s, 0)),
            pl.BlockSpec((1, 1, C, D), lambda mm, b, s: (mm, b, 0, 0)),
            pl.BlockSpec((1, 1, 1, D), lambda mm, b, s: (mm, b, 0, 0)),
            pl.BlockSpec((1, 1, D, C), lambda mm, b, s: (mm, b, 0, 0)),
        ],
        out_specs=pl.BlockSpec((1, 1, TS, C), lambda mm, b, s: (mm, b, s, 0)),
        out_shape=jax.ShapeDtypeStruct((M, B, S, C), jnp.float32),
        compiler_params=pltpu.CompilerParams(
            dimension_semantics=("parallel", "parallel", "parallel"),
        ),
    )(x, bdw, bdb, buw)
    return out


# speedup vs baseline: 1.5578x; 1.0357x over previous
import functools

import jax
import jax.numpy as jnp
from jax.experimental import pallas as pl
from jax.experimental.pallas import tpu as pltpu


def _adapter_body(x_ref, dw_ref, db_ref, uw_ref, o_ref):
    x = x_ref[0]
    dw = dw_ref[0, 0]
    db = db_ref[0, 0, 0]
    uw = uw_ref[0, 0]
    z = jnp.dot(x, dw, preferred_element_type=jnp.float32) + db[None, :]
    z = z * jax.nn.sigmoid(z)
    o_ref[0, 0] = jnp.dot(z, uw, preferred_element_type=jnp.float32)


@jax.jit
def kernel(x, expert_index, down_w, down_b, up_w):
    B, S, C = x.shape
    M, N, _, D = down_w.shape
    TS = 1024
    s_blocks = S // TS

    idx = expert_index.astype(jnp.int32)
    m = jnp.arange(M)[:, None]
    bdw = down_w[m, idx]
    bdb = down_b[m, idx].reshape(M, B, 1, D)
    buw = up_w[m, idx]

    grid = (M, B, s_blocks)

    out = pl.pallas_call(
        _adapter_body,
        grid=grid,
        in_specs=[
            pl.BlockSpec((1, TS, C), lambda mm, b, s: (b, s, 0)),
            pl.BlockSpec((1, 1, C, D), lambda mm, b, s: (mm, b, 0, 0)),
            pl.BlockSpec((1, 1, 1, D), lambda mm, b, s: (mm, b, 0, 0)),
            pl.BlockSpec((1, 1, D, C), lambda mm, b, s: (mm, b, 0, 0)),
        ],
        out_specs=pl.BlockSpec((1, 1, TS, C), lambda mm, b, s: (mm, b, s, 0)),
        out_shape=jax.ShapeDtypeStruct((M, B, S, C), jnp.float32),
        compiler_params=pltpu.CompilerParams(
            dimension_semantics=("parallel", "parallel", "parallel"),
        ),
    )(x, bdw, bdb, buw)
    return out
